# Initial kernel scaffold; baseline (speedup 1.0000x reference)
import jax
import jax.numpy as jnp
from jax.experimental import pallas as pl


def kernel(table, embeddings_idx):
    raise NotImplementedError("write your pallas kernel here")


# speedup vs baseline: 1.3033x; 1.3033x over previous
import functools

import jax
import jax.numpy as jnp
from jax import lax
from jax.experimental import pallas as pl
from jax.experimental.pallas import tpu as pltpu
from jax.experimental.pallas import tpu_sc as plsc

_DIM = 128
_N = 4880
_NC = 2
_NS = 16
_NW = _NC * _NS
_CHUNK = 80
_NCHUNK = _N // _CHUNK

_mesh = plsc.VectorSubcoreMesh(core_axis_name="c", subcore_axis_name="s")


@functools.partial(
    pl.kernel,
    out_type=jax.ShapeDtypeStruct((_N, _DIM), jnp.float32),
    mesh=_mesh,
    scratch_types=[
        pltpu.VMEM((2, _CHUNK), jnp.int32),
        pltpu.VMEM((2, _CHUNK, _DIM), jnp.float32),
        pltpu.SemaphoreType.DMA,
        pltpu.SemaphoreType.DMA,
    ],
)
def _gather(table_hbm, idx_hbm, out_hbm, idx_v, rows_v, sem0, sem1):
    wid = lax.axis_index("s") * _NC + lax.axis_index("c")
    c0 = wid
    c1 = jnp.minimum(wid + _NW, _NCHUNK - 1)
    b0 = c0 * _CHUNK
    b1 = c1 * _CHUNK
    pltpu.sync_copy(idx_hbm.at[pl.ds(b0, _CHUNK)], idx_v.at[0])
    pltpu.sync_copy(idx_hbm.at[pl.ds(b1, _CHUNK)], idx_v.at[1])
    g0 = pltpu.async_copy(table_hbm.at[idx_v.at[0]], rows_v.at[0], sem0)
    g1 = pltpu.async_copy(table_hbm.at[idx_v.at[1]], rows_v.at[1], sem1)
    g0.wait()
    pltpu.sync_copy(rows_v.at[0], out_hbm.at[pl.ds(b0, _CHUNK)])
    g1.wait()
    pltpu.sync_copy(rows_v.at[1], out_hbm.at[pl.ds(b1, _CHUNK)])


def kernel(table, embeddings_idx):
    return _gather(table, embeddings_idx)
